# baseline (speedup 1.0000x reference)
import jax
import jax.numpy as jnp
from jax.experimental import pallas as pl
from jax.experimental.pallas import tpu as pltpu

_E, _K, _R, _ALPHA = 8, 2, 8, 8
_SCALING = _ALPHA / _R
_TILE = 512
_NT = 16


def _compute_tile(xb, p, idx, wd, wu):
    down = jnp.dot(xb, wd, preferred_element_type=jnp.float32)
    act = 0.5 * down * (1.0 + jax.lax.erf(down * 0.7071067811865476))
    tile, er = act.shape
    eidx = jax.lax.broadcasted_iota(jnp.int32, (tile, er), 1) // _R
    wrep = jnp.zeros((tile, er), jnp.float32)
    for k in range(_K):
        wrep = wrep + jnp.where(idx[:, k][:, None] == eidx,
                                p[:, k][:, None], 0.0)
    up = jnp.dot(act * wrep, wu, preferred_element_type=jnp.float32)
    return xb + up * _SCALING


def _moe_lora_kernel(p_ref, i_ref, wd_ref, wu_ref, x_hbm, o_hbm,
                     xbuf, obuf, insem, outsem):
    def in_copy(i, slot):
        return pltpu.make_async_copy(
            x_hbm.at[pl.ds(i * _TILE, _TILE), :], xbuf.at[slot],
            insem.at[slot])

    def out_copy(i, slot):
        return pltpu.make_async_copy(
            obuf.at[slot], o_hbm.at[pl.ds(i * _TILE, _TILE), :],
            outsem.at[slot])

    in_copy(0, 0).start()
    for i in range(_NT):
        slot = i % 2
        if i + 1 < _NT:
            in_copy(i + 1, (i + 1) % 2).start()
        in_copy(i, slot).wait()
        if i >= 2:
            out_copy(i - 2, slot).wait()
        obuf[slot] = _compute_tile(
            xbuf[slot],
            p_ref[pl.ds(i * _TILE, _TILE), :],
            i_ref[pl.ds(i * _TILE, _TILE), :],
            wd_ref[...], wu_ref[...])
        out_copy(i, slot).start()
    out_copy(_NT - 2, (_NT - 2) % 2).wait()
    out_copy(_NT - 1, (_NT - 1) % 2).wait()


@jax.jit
def kernel(x, gate_probs, topk_probs, topk_indices, w_down, w_up):
    del gate_probs
    b, s, dim = x.shape
    t = b * s
    e, _, r = w_down.shape
    x_flat = x.reshape(t, dim)
    wd = jnp.transpose(w_down, (1, 0, 2)).reshape(dim, e * r)
    wu = w_up.reshape(e * r, dim)
    topk_indices = topk_indices.astype(jnp.int32)

    out = pl.pallas_call(
        _moe_lora_kernel,
        in_specs=[
            pl.BlockSpec(memory_space=pltpu.MemorySpace.VMEM),
            pl.BlockSpec(memory_space=pltpu.MemorySpace.VMEM),
            pl.BlockSpec(memory_space=pltpu.MemorySpace.VMEM),
            pl.BlockSpec(memory_space=pltpu.MemorySpace.VMEM),
            pl.BlockSpec(memory_space=pltpu.MemorySpace.HBM),
        ],
        out_specs=pl.BlockSpec(memory_space=pltpu.MemorySpace.HBM),
        out_shape=jax.ShapeDtypeStruct((t, dim), jnp.float32),
        scratch_shapes=[
            pltpu.VMEM((2, _TILE, dim), jnp.float32),
            pltpu.VMEM((2, _TILE, dim), jnp.float32),
            pltpu.SemaphoreType.DMA((2,)),
            pltpu.SemaphoreType.DMA((2,)),
        ],
    )(topk_probs, topk_indices, wd, wu, x_flat)
    return out.reshape(b, s, dim)


# speedup vs baseline: 1.0619x; 1.0619x over previous
import jax
import jax.numpy as jnp
from jax.experimental import pallas as pl
from jax.experimental.pallas import tpu as pltpu

_E, _K, _R, _ALPHA = 8, 2, 8, 8
_SCALING = _ALPHA / _R
_TILE = 1024


def _moe_lora_kernel(x_ref, p_ref, i_ref, wd_ref, wu_ref, o_ref):
    xb = x_ref[...]
    down = jnp.dot(xb, wd_ref[...],
                   preferred_element_type=jnp.float32)
    act = 0.5 * down * (1.0 + jax.lax.erf(down * 0.7071067811865476))

    tile, er = act.shape
    eidx = jax.lax.broadcasted_iota(jnp.int32, (tile, er), 1) // _R
    wrep = jnp.zeros((tile, er), jnp.float32)
    for k in range(_K):
        idx_k = i_ref[:, k][:, None]
        p_k = p_ref[:, k][:, None]
        wrep = wrep + jnp.where(idx_k == eidx, p_k, 0.0)

    up = jnp.dot(act * wrep, wu_ref[...],
                 preferred_element_type=jnp.float32)
    o_ref[...] = xb + up * _SCALING


@jax.jit
def kernel(x, gate_probs, topk_probs, topk_indices, w_down, w_up):
    del gate_probs
    b, s, dim = x.shape
    t = b * s
    e, _, r = w_down.shape
    x_flat = x.reshape(t, dim)
    wd = jnp.transpose(w_down, (1, 0, 2)).reshape(dim, e * r)
    wu = w_up.reshape(e * r, dim)
    topk_indices = topk_indices.astype(jnp.int32)

    grid = (t // _TILE,)
    out = pl.pallas_call(
        _moe_lora_kernel,
        grid=grid,
        in_specs=[
            pl.BlockSpec((_TILE, dim), lambda i: (i, 0)),
            pl.BlockSpec((_TILE, _K), lambda i: (i, 0)),
            pl.BlockSpec((_TILE, _K), lambda i: (i, 0)),
            pl.BlockSpec((dim, e * r), lambda i: (0, 0)),
            pl.BlockSpec((e * r, dim), lambda i: (0, 0)),
        ],
        out_specs=pl.BlockSpec((_TILE, dim), lambda i: (i, 0)),
        out_shape=jax.ShapeDtypeStruct((t, dim), jnp.float32),
        compiler_params=pltpu.CompilerParams(
            dimension_semantics=("parallel",)),
    )(x_flat, topk_probs, topk_indices, wd, wu)
    return out.reshape(b, s, dim)
